# baseline (speedup 1.0000x reference)
import functools

import jax
import jax.numpy as jnp
from jax import lax
from jax.experimental import pallas as pl
from jax.experimental.pallas import tpu as pltpu
from jax.experimental.pallas import tpu_sc as plsc

SEQ = 200
EMB = 64
OUT = 2
BATCH = 4096
NC, NS = 2, 16
NW = NC * NS
CPT = BATCH // NW


def _fasttext_body(textt_hbm, table_hbm, w_hbm, b_hbm, out_hbm,
                   idx_v, rows_v, w_v, b_v, pooled_v, out_v, sem):
    wid = lax.axis_index("s") * NC + lax.axis_index("c")
    c0 = wid * CPT

    pltpu.sync_copy(textt_hbm.at[pl.ds(c0, CPT), :], idx_v)
    pltpu.sync_copy(w_hbm, w_v)
    pltpu.sync_copy(b_hbm, b_v.at[pl.ds(0, OUT)])

    inv = jnp.float32(1.0 / SEQ)
    w_regs = [[w_v[o, pl.ds(k * 16, 16)] * inv for k in range(4)]
              for o in range(2)]
    lane = lax.iota(jnp.int32, 16)
    bvec = b_v[pl.ds(0, 16)]

    zero = jnp.zeros((16,), jnp.float32)

    def col_body(c, carry):
        cp1 = pltpu.async_copy(
            table_hbm.at[idx_v.at[c, pl.ds(0, 128)]],
            rows_v.at[pl.ds(0, 128), :], sem)
        cp2 = pltpu.async_copy(
            table_hbm.at[idx_v.at[c, pl.ds(128, SEQ - 128)]],
            rows_v.at[pl.ds(128, SEQ - 128), :], sem)
        cp1.wait()
        cp2.wait()

        def seq_body(i, accs):
            a0, a1, a2, a3 = accs
            for j in range(8):
                r = i * 8 + j
                a0 = a0 + rows_v[r, pl.ds(0, 16)]
                a1 = a1 + rows_v[r, pl.ds(16, 16)]
                a2 = a2 + rows_v[r, pl.ds(32, 16)]
                a3 = a3 + rows_v[r, pl.ds(48, 16)]
            return a0, a1, a2, a3

        accs = lax.fori_loop(0, SEQ // 8, seq_body, (zero, zero, zero, zero))
        for k in range(4):
            pooled_v[pl.ds(c * EMB + k * 16, 16)] = accs[k]
        return carry

    lax.fori_loop(0, CPT, col_body, 0)

    for g in range(CPT // 16):
        acc0 = zero
        acc1 = zero
        base = g * 16 * EMB
        for e in range(EMB):
            eidx = lane * EMB + (base + e)
            v = plsc.load_gather(pooled_v, [eidx])
            w0e = w_regs[0][e // 16][e % 16]
            w1e = w_regs[1][e // 16][e % 16]
            acc0 = acc0 + v * w0e
            acc1 = acc1 + v * w1e
        o_base = g * 32
        plsc.store_scatter(out_v, [lane * 2 + o_base], acc0 + bvec[0])
        plsc.store_scatter(out_v, [lane * 2 + (o_base + 1)], acc1 + bvec[1])

    pltpu.sync_copy(out_v, out_hbm.at[pl.ds(wid * CPT * OUT, CPT * OUT)])


@functools.partial(
    pl.kernel,
    out_type=jax.ShapeDtypeStruct((BATCH * OUT,), jnp.float32),
    mesh=plsc.VectorSubcoreMesh(core_axis_name="c", subcore_axis_name="s",
                                num_cores=NC, num_subcores=NS),
    compiler_params=pltpu.CompilerParams(needs_layout_passes=False,
                                         use_tc_tiling_on_sc=False),
    scratch_types=[
        pltpu.VMEM((CPT, SEQ), jnp.int32),
        pltpu.VMEM((SEQ, EMB), jnp.float32),
        pltpu.VMEM((OUT, EMB), jnp.float32),
        pltpu.VMEM((16,), jnp.float32),
        pltpu.VMEM((CPT * EMB,), jnp.float32),
        pltpu.VMEM((CPT * OUT,), jnp.float32),
        pltpu.SemaphoreType.DMA,
    ],
)
def _fasttext_sc(textt, table, w, b, out,
                 idx_v, rows_v, w_v, b_v, pooled_v, out_v, sem):
    _fasttext_body(textt, table, w, b, out,
                   idx_v, rows_v, w_v, b_v, pooled_v, out_v, sem)


def kernel(text, table, W, b):
    textt = text.T
    out_flat = _fasttext_sc(textt, table, W, b)
    return out_flat.reshape(BATCH, OUT)


# speedup vs baseline: 1.1951x; 1.1951x over previous
import functools

import jax
import jax.numpy as jnp
from jax import lax
from jax.experimental import pallas as pl
from jax.experimental.pallas import tpu as pltpu
from jax.experimental.pallas import tpu_sc as plsc

SEQ = 200
EMB = 64
OUT = 2
BATCH = 4096
NC, NS = 2, 16
NW = NC * NS
CPT = BATCH // NW


def _fasttext_body(textt_hbm, table_hbm, w_hbm, b_hbm, out_hbm,
                   idx_v, rows0_v, rows1_v, rows2_v, rows3_v,
                   w_v, b_v, pooled_v, out_v, sem0, sem1, sem2, sem3):
    wid = lax.axis_index("s") * NC + lax.axis_index("c")
    c0 = wid * CPT

    pltpu.sync_copy(textt_hbm.at[pl.ds(c0, CPT), :], idx_v)
    pltpu.sync_copy(w_hbm, w_v)
    pltpu.sync_copy(b_hbm, b_v.at[pl.ds(0, OUT)])

    inv = jnp.float32(1.0 / SEQ)
    w_regs = [[w_v[o, pl.ds(k * 16, 16)] * inv for k in range(4)]
              for o in range(2)]
    lane = lax.iota(jnp.int32, 16)
    bvec = b_v[pl.ds(0, 16)]

    zero = jnp.zeros((16,), jnp.float32)
    bufs = (rows0_v, rows1_v, rows2_v, rows3_v)
    sems = (sem0, sem1, sem2, sem3)

    def issue(c, buf, s):
        pltpu.async_copy(table_hbm.at[idx_v.at[c, pl.ds(0, 128)]],
                         buf.at[pl.ds(0, 128), :], s)
        pltpu.async_copy(table_hbm.at[idx_v.at[c, pl.ds(128, SEQ - 128)]],
                         buf.at[pl.ds(128, SEQ - 128), :], s)

    def drain(buf, s):
        pltpu.make_async_copy(table_hbm.at[pl.ds(0, SEQ), :], buf, s).wait()

    def process(c, buf):
        def seq_body(i, accs):
            a0, a1, a2, a3, b0, b1, b2, b3 = accs
            for j in range(0, 8, 2):
                r = i * 8 + j
                a0 = a0 + buf[r, pl.ds(0, 16)]
                a1 = a1 + buf[r, pl.ds(16, 16)]
                a2 = a2 + buf[r, pl.ds(32, 16)]
                a3 = a3 + buf[r, pl.ds(48, 16)]
                b0 = b0 + buf[r + 1, pl.ds(0, 16)]
                b1 = b1 + buf[r + 1, pl.ds(16, 16)]
                b2 = b2 + buf[r + 1, pl.ds(32, 16)]
                b3 = b3 + buf[r + 1, pl.ds(48, 16)]
            return a0, a1, a2, a3, b0, b1, b2, b3

        accs = lax.fori_loop(0, SEQ // 8, seq_body, (zero,) * 8)
        for k in range(4):
            pooled_v[pl.ds(c * EMB + k * 16, 16)] = accs[k] + accs[k + 4]

    for b in range(3):
        issue(b, bufs[b], sems[b])

    def col_group(g, carry):
        for j in range(4):
            c = 4 * g + j
            ahead = c + 3
            jn = (j + 3) % 4

            @pl.when(ahead < CPT)
            def _():
                issue(ahead, bufs[jn], sems[jn])

            drain(bufs[j], sems[j])
            process(c, bufs[j])
        return carry

    lax.fori_loop(0, CPT // 4, col_group, 0)

    for g in range(CPT // 16):
        acc0 = zero
        acc1 = zero
        base = g * 16 * EMB
        for e in range(EMB):
            eidx = lane * EMB + (base + e)
            v = plsc.load_gather(pooled_v, [eidx])
            w0e = w_regs[0][e // 16][e % 16]
            w1e = w_regs[1][e // 16][e % 16]
            acc0 = acc0 + v * w0e
            acc1 = acc1 + v * w1e
        o_base = g * 32
        plsc.store_scatter(out_v, [lane * 2 + o_base], acc0 + bvec[0])
        plsc.store_scatter(out_v, [lane * 2 + (o_base + 1)], acc1 + bvec[1])

    pltpu.sync_copy(out_v, out_hbm.at[pl.ds(wid * CPT * OUT, CPT * OUT)])


@functools.partial(
    pl.kernel,
    out_type=jax.ShapeDtypeStruct((BATCH * OUT,), jnp.float32),
    mesh=plsc.VectorSubcoreMesh(core_axis_name="c", subcore_axis_name="s",
                                num_cores=NC, num_subcores=NS),
    compiler_params=pltpu.CompilerParams(needs_layout_passes=False,
                                         use_tc_tiling_on_sc=False),
    scratch_types=[
        pltpu.VMEM((CPT, SEQ), jnp.int32),
        pltpu.VMEM((SEQ, EMB), jnp.float32),
        pltpu.VMEM((SEQ, EMB), jnp.float32),
        pltpu.VMEM((SEQ, EMB), jnp.float32),
        pltpu.VMEM((SEQ, EMB), jnp.float32),
        pltpu.VMEM((OUT, EMB), jnp.float32),
        pltpu.VMEM((16,), jnp.float32),
        pltpu.VMEM((CPT * EMB,), jnp.float32),
        pltpu.VMEM((CPT * OUT,), jnp.float32),
        pltpu.SemaphoreType.DMA,
        pltpu.SemaphoreType.DMA,
        pltpu.SemaphoreType.DMA,
        pltpu.SemaphoreType.DMA,
    ],
)
def _fasttext_sc(textt, table, w, b, out,
                 idx_v, rows0_v, rows1_v, rows2_v, rows3_v,
                 w_v, b_v, pooled_v, out_v, sem0, sem1, sem2, sem3):
    _fasttext_body(textt, table, w, b, out,
                   idx_v, rows0_v, rows1_v, rows2_v, rows3_v,
                   w_v, b_v, pooled_v, out_v, sem0, sem1, sem2, sem3)


def kernel(text, table, W, b):
    textt = text.T
    out_flat = _fasttext_sc(textt, table, W, b)
    return out_flat.reshape(BATCH, OUT)
